# baseline (speedup 1.0000x reference)
import functools

import jax
import jax.numpy as jnp
from jax import lax
from jax.experimental import pallas as pl
from jax.experimental.pallas import tpu as pltpu
from jax.experimental.pallas import tpu_sc as plsc

_N, _H, _W, _K = 4, 512, 512, 4
_NPIX = _N * _H * _W
_L = 16
_NC, _NS = 2, 16
_NW = _NC * _NS
_B = 4096
_CPW = _NPIX // (_B * _NW)


def _sc_body(p2f_hbm, tex_hbm, bg_hbm, out_hbm, p2f_v, tex_v, out_v, bg_v):
    wid = lax.axis_index("s") * _NC + lax.axis_index("c")

    lane = lax.iota(jnp.int32, _L)
    quad = lane >> 2
    chan = lane & 3
    pat_mask = quad << 2
    pat_tex = quad * 12 + chan
    alpha_lane = chan == 3

    pltpu.sync_copy(bg_hbm, bg_v)
    bg_vec = bg_v[...]
    one = jnp.float32(1.0)

    def do_chunk(c, _):
        gbase = (wid * _CPW + c) * _B
        pltpu.sync_copy(p2f_hbm.at[pl.ds(pl.multiple_of(gbase * 4, 8), _B * 4)], p2f_v)
        pltpu.sync_copy(tex_hbm.at[pl.ds(pl.multiple_of(gbase * 12, 8), _B * 12)], tex_v)

        def do_vreg(j, _):
            b4 = j * _L
            gm = plsc.load_gather(p2f_v, [pat_mask + b4])
            gt = plsc.load_gather(tex_v, [pat_tex + j * (3 * _L)])
            fg = jnp.where(alpha_lane, one, gt)
            out_v[pl.ds(b4, _L)] = jnp.where(gm < 0, bg_vec, fg)
            return 0

        lax.fori_loop(0, _B // 4, do_vreg, 0)
        pltpu.sync_copy(out_v, out_hbm.at[pl.ds(pl.multiple_of(gbase * 4, 8), _B * 4)])
        return 0

    lax.fori_loop(0, _CPW, do_chunk, 0)


@jax.jit
def _shade(p2f_flat, tex_flat, bg16):
    k = pl.kernel(
        _sc_body,
        out_type=jax.ShapeDtypeStruct((_NPIX * 4,), jnp.float32),
        mesh=plsc.VectorSubcoreMesh(core_axis_name="c", subcore_axis_name="s"),
        compiler_params=pltpu.CompilerParams(needs_layout_passes=False),
        scratch_types=[
            pltpu.VMEM((_B * 4,), jnp.int32),
            pltpu.VMEM((_B * 12,), jnp.float32),
            pltpu.VMEM((_B * 4,), jnp.float32),
            pltpu.VMEM((_L,), jnp.float32),
        ],
    )
    return k(p2f_flat, tex_flat, bg16)


def kernel(verts, faces, face_normals, pix_to_face, texels, background_color):
    del verts, faces, face_normals
    p2f_flat = pix_to_face.reshape(-1)
    tex_flat = texels.reshape(-1)
    bg4 = jnp.concatenate([background_color.astype(jnp.float32),
                           jnp.zeros((1,), jnp.float32)])
    bg16 = jnp.tile(bg4, 4)
    out = _shade(p2f_flat, tex_flat, bg16)
    return out.reshape(_N, _H, _W, 4)


# speedup vs baseline: 292.3091x; 292.3091x over previous
import functools

import jax
import jax.numpy as jnp
from jax import lax
from jax.experimental import pallas as pl
from jax.experimental.pallas import tpu as pltpu
from jax.experimental.pallas import tpu_sc as plsc

_N, _H, _W, _K = 4, 512, 512, 4
_WT = _W // 128
_TH = _N * _H
_NT = _TH * _WT
_L = 16
_NC, _NS = 2, 16
_NW = _NC * _NS
_TPW = _NT // _NW
_CT = 64
_CTH = _CT // _WT
_NCH = _TPW // _CT


def _sc_body(p2f_hbm, tex_hbm, bg_hbm, out_hbm, p2f_v, tex_v, out_v, bg_v):
    wid = lax.axis_index("s") * _NC + lax.axis_index("c")

    pltpu.sync_copy(bg_hbm, bg_v)
    bg0 = bg_v[pl.ds(0, _L)]
    bg1 = bg_v[pl.ds(16, _L)]
    bg2 = bg_v[pl.ds(32, _L)]
    zero = jnp.zeros((_L,), jnp.float32)
    one = jnp.full((_L,), 1.0, jnp.float32)

    def do_chunk(ch, _):
        t0 = wid * _TPW + ch * _CT
        th0 = t0 // _WT
        pltpu.sync_copy(p2f_hbm.at[pl.ds(t0, _CT), pl.ds(0, 1), :], p2f_v)
        for c in range(3):
            pltpu.sync_copy(
                tex_hbm.at[pl.ds(th0, _CTH), pl.ds(c, 1), :, pl.ds(0, 1), :],
                tex_v.at[:, pl.ds(c, 1)],
            )

        def do_tile(j, _):
            thj = j >> 2
            wtj = j & 3
            for l in range(8):
                m = p2f_v[j, 0, pl.ds(l * _L, _L)] < 0
                r = tex_v[thj, 0, wtj, 0, pl.ds(l * _L, _L)]
                g = tex_v[thj, 1, wtj, 0, pl.ds(l * _L, _L)]
                b = tex_v[thj, 2, wtj, 0, pl.ds(l * _L, _L)]
                out_v[j, 0, pl.ds(l * _L, _L)] = jnp.where(m, bg0, r)
                out_v[j, 1, pl.ds(l * _L, _L)] = jnp.where(m, bg1, g)
                out_v[j, 2, pl.ds(l * _L, _L)] = jnp.where(m, bg2, b)
                out_v[j, 3, pl.ds(l * _L, _L)] = jnp.where(m, zero, one)
            return 0

        lax.fori_loop(0, _CT, do_tile, 0)
        pltpu.sync_copy(out_v, out_hbm.at[pl.ds(t0, _CT)])
        return 0

    lax.fori_loop(0, _NCH, do_chunk, 0)


@jax.jit
def _shade(p2f_t, tex_t, bg48):
    k = pl.kernel(
        _sc_body,
        out_type=jax.ShapeDtypeStruct((_NT, 4, 128), jnp.float32),
        mesh=plsc.VectorSubcoreMesh(core_axis_name="c", subcore_axis_name="s"),
        compiler_params=pltpu.CompilerParams(needs_layout_passes=False),
        scratch_types=[
            pltpu.VMEM((_CT, 1, 128), jnp.int32),
            pltpu.VMEM((_CTH, 3, _WT, 1, 128), jnp.float32),
            pltpu.VMEM((_CT, 4, 128), jnp.float32),
            pltpu.VMEM((48,), jnp.float32),
        ],
    )
    return k(p2f_t, tex_t, bg48)


def kernel(verts, faces, face_normals, pix_to_face, texels, background_color):
    del verts, faces, face_normals
    p2f_t = (pix_to_face.reshape(_N, _H, _WT, 128, _K)
             .transpose(0, 1, 2, 4, 3)
             .reshape(_NT, 4, 128))
    tex_t = (texels.reshape(_N, _H, _WT, 128, _K, 3)
             .transpose(0, 1, 5, 2, 4, 3)
             .reshape(_TH, 3, _WT, 4, 128))
    bg48 = jnp.repeat(background_color.astype(jnp.float32), _L)
    out = _shade(p2f_t, tex_t, bg48)
    return (out.reshape(_N, _H, _WT, 4, 128)
            .transpose(0, 1, 2, 4, 3)
            .reshape(_N, _H, _W, 4))


# speedup vs baseline: 402.1092x; 1.3756x over previous
import functools

import jax
import jax.numpy as jnp
from jax import lax
from jax.experimental import pallas as pl
from jax.experimental.pallas import tpu as pltpu
from jax.experimental.pallas import tpu_sc as plsc

_N, _H, _W, _K = 4, 512, 512, 4
_WT = _W // 128
_TH = _N * _H
_NT = _TH * _WT
_L = 16
_NC, _NS = 2, 16
_NW = _NC * _NS
_TPW = _NT // _NW
_CT = 32
_CTH = _CT // _WT
_NCH = _TPW // _CT


def _sc_body(p2f_hbm, tex_hbm, bg_hbm, out_hbm,
             p2f_v, tex_v, out_v, bg_v, insem, outsem):
    wid = lax.axis_index("s") * _NC + lax.axis_index("c")
    base = wid * _TPW

    pltpu.sync_copy(bg_hbm, bg_v)
    bg0 = bg_v[pl.ds(0, _L)]
    bg1 = bg_v[pl.ds(16, _L)]
    bg2 = bg_v[pl.ds(32, _L)]
    zero = jnp.zeros((_L,), jnp.float32)
    one = jnp.full((_L,), 1.0, jnp.float32)

    def start_in(ch):
        par = ch & 1
        t0 = base + ch * _CT
        th0 = t0 // _WT
        descs = [pltpu.async_copy(
            p2f_hbm.at[pl.ds(t0, _CT), pl.ds(0, 1), :],
            p2f_v.at[par], insem.at[par])]
        for c in range(3):
            descs.append(pltpu.async_copy(
                tex_hbm.at[pl.ds(th0, _CTH), pl.ds(c, 1), :, pl.ds(0, 1), :],
                tex_v.at[par, :, pl.ds(c, 1)], insem.at[par]))
        return descs

    def compute(ch):
        par = ch & 1

        def do_tile(j, _):
            thj = j >> 2
            wtj = j & 3
            for l in range(8):
                m = p2f_v[par, j, 0, pl.ds(l * _L, _L)] < 0
                r = tex_v[par, thj, 0, wtj, 0, pl.ds(l * _L, _L)]
                g = tex_v[par, thj, 1, wtj, 0, pl.ds(l * _L, _L)]
                b = tex_v[par, thj, 2, wtj, 0, pl.ds(l * _L, _L)]
                out_v[par, j, 0, pl.ds(l * _L, _L)] = jnp.where(m, bg0, r)
                out_v[par, j, 1, pl.ds(l * _L, _L)] = jnp.where(m, bg1, g)
                out_v[par, j, 2, pl.ds(l * _L, _L)] = jnp.where(m, bg2, b)
                out_v[par, j, 3, pl.ds(l * _L, _L)] = jnp.where(m, zero, one)
            return 0

        lax.fori_loop(0, _CT, do_tile, 0)

    in_descs = {0: start_in(0)}
    out_descs = {}
    for ch in range(_NCH):
        par = ch & 1
        if ch + 1 < _NCH:
            in_descs[ch + 1] = start_in(ch + 1)
        for d in in_descs.pop(ch):
            d.wait()
        if ch >= 2:
            out_descs.pop(ch - 2).wait()
        compute(ch)
        out_descs[ch] = pltpu.async_copy(
            out_v.at[par], out_hbm.at[pl.ds(base + ch * _CT, _CT)],
            outsem.at[par])
    for ch in sorted(out_descs):
        out_descs[ch].wait()


@jax.jit
def _shade(p2f_t, tex_t, bg48):
    k = pl.kernel(
        _sc_body,
        out_type=jax.ShapeDtypeStruct((_NT, 4, 128), jnp.float32),
        mesh=plsc.VectorSubcoreMesh(core_axis_name="c", subcore_axis_name="s"),
        compiler_params=pltpu.CompilerParams(needs_layout_passes=False),
        scratch_types=[
            pltpu.VMEM((2, _CT, 1, 128), jnp.int32),
            pltpu.VMEM((2, _CTH, 3, _WT, 1, 128), jnp.float32),
            pltpu.VMEM((2, _CT, 4, 128), jnp.float32),
            pltpu.VMEM((48,), jnp.float32),
            pltpu.SemaphoreType.DMA((2,)),
            pltpu.SemaphoreType.DMA((2,)),
        ],
    )
    return k(p2f_t, tex_t, bg48)


def kernel(verts, faces, face_normals, pix_to_face, texels, background_color):
    del verts, faces, face_normals
    p2f_t = (pix_to_face.reshape(_N, _H, _WT, 128, _K)
             .transpose(0, 1, 2, 4, 3)
             .reshape(_NT, 4, 128))
    tex_t = (texels.reshape(_N, _H, _WT, 128, _K, 3)
             .transpose(0, 1, 5, 2, 4, 3)
             .reshape(_TH, 3, _WT, 4, 128))
    bg48 = jnp.repeat(background_color.astype(jnp.float32), _L)
    out = _shade(p2f_t, tex_t, bg48)
    return (out.reshape(_N, _H, _WT, 4, 128)
            .transpose(0, 1, 2, 4, 3)
            .reshape(_N, _H, _W, 4))
